# Initial kernel scaffold; baseline (speedup 1.0000x reference)
import jax
import jax.numpy as jnp
from jax.experimental import pallas as pl


def kernel(x, edge_index, W1, b1, W2, b2):
    raise NotImplementedError("write your pallas kernel here")


# speedup vs baseline: 11.5321x; 11.5321x over previous
import jax
import jax.numpy as jnp
from jax import lax
from jax.experimental import pallas as pl
from jax.experimental.pallas import tpu as pltpu
from jax.experimental.pallas import tpu_sc as plsc

N = 10000
E = 320000
D_IN = 128
D_HID = 128
D_OUT = 64

NP = 10240
RPT = NP // 16
CH = 128
NTILES = 32
EP = 323584
EPT = EP // NTILES
NCH = EPT // CH
BR = 512


def _make_deg():
    mesh = plsc.VectorSubcoreMesh(core_axis_name="c", subcore_axis_name="s")

    def body(dst_hbm, z_hbm, out_hbm, idx_d, ones, table):
        cid = lax.axis_index("c")
        sid = lax.axis_index("s")
        r0 = sid * RPT
        pltpu.sync_copy(z_hbm.at[pl.ds(r0, RPT)], table.at[pl.ds(r0, RPT)])
        for j in range(CH // 16):
            ones[pl.ds(j * 16, 16)] = jnp.ones((16,), jnp.float32)
        plsc.subcore_barrier()
        ebase = (cid * 16 + sid) * EPT

        def step(i, carry):
            e0 = ebase + i * CH
            pltpu.sync_copy(dst_hbm.at[pl.ds(e0, CH)], idx_d)
            pltpu.sync_copy(ones, table.at[idx_d], add=True)
            return carry

        lax.fori_loop(0, NCH, step, 0)
        plsc.subcore_barrier()
        pltpu.sync_copy(table.at[pl.ds(r0, RPT)], out_hbm.at[cid, pl.ds(r0, RPT)])

    return pl.kernel(
        body,
        out_type=jax.ShapeDtypeStruct((2, NP), jnp.float32),
        mesh=mesh,
        scratch_types=[
            pltpu.VMEM((CH,), jnp.int32),
            pltpu.VMEM((CH,), jnp.float32),
            pltpu.VMEM_SHARED((NP,), jnp.float32),
        ],
    )


def _make_agg(d):
    mesh = plsc.VectorSubcoreMesh(core_axis_name="c", subcore_axis_name="s")

    def body(y_hbm, src_hbm, dst_hbm, z_hbm, out_hbm, idx_s, idx_d, rows, table, sem):
        cid = lax.axis_index("c")
        sid = lax.axis_index("s")
        r0 = sid * RPT
        pltpu.sync_copy(z_hbm.at[pl.ds(r0, RPT)], table.at[pl.ds(r0, RPT)])
        plsc.subcore_barrier()
        ebase = (cid * 16 + sid) * EPT

        def step(i, carry):
            e0 = ebase + i * CH
            pltpu.sync_copy(src_hbm.at[pl.ds(e0, CH)], idx_s)
            pltpu.sync_copy(dst_hbm.at[pl.ds(e0, CH)], idx_d)
            pltpu.async_copy(y_hbm.at[idx_s], rows, sem).wait()
            pltpu.sync_copy(rows, table.at[idx_d], add=True)
            return carry

        lax.fori_loop(0, NCH, step, 0)
        plsc.subcore_barrier()
        pltpu.sync_copy(table.at[pl.ds(r0, RPT)], out_hbm.at[cid, pl.ds(r0, RPT)])

    return pl.kernel(
        body,
        out_type=jax.ShapeDtypeStruct((2, NP, d), jnp.float32),
        mesh=mesh,
        scratch_types=[
            pltpu.VMEM((CH,), jnp.int32),
            pltpu.VMEM((CH,), jnp.int32),
            pltpu.VMEM((CH, d), jnp.float32),
            pltpu.VMEM_SHARED((NP, d), jnp.float32),
            pltpu.SemaphoreType.DMA,
        ],
    )


_deg = _make_deg()
_agg_hid = _make_agg(D_HID)


def _dinv_of(dp_ref):
    deg = dp_ref[0, :] + dp_ref[1, :] + 1.0
    return lax.rsqrt(jnp.maximum(deg, 1e-12))


def _mm1_body(dp_ref, x_ref, w_ref, o_ref):
    dinv = _dinv_of(dp_ref)
    xw = jnp.dot(x_ref[...], w_ref[...], preferred_element_type=jnp.float32)
    o_ref[...] = xw * dinv[:, None]


def _mm2_body(dp_ref, a_ref, y_ref, b_ref, w_ref, o_ref):
    dinv = _dinv_of(dp_ref)
    h = (a_ref[0] + a_ref[1] + y_ref[...]) * dinv[:, None] + b_ref[...]
    h = jnp.maximum(h, 0.0)
    hw = jnp.dot(h, w_ref[...], preferred_element_type=jnp.float32)
    o_ref[...] = hw * dinv[:, None]


def _fin_body(dp_ref, a_ref, y_ref, b_ref, o_ref):
    dinv = _dinv_of(dp_ref)
    o_ref[...] = (a_ref[0] + a_ref[1] + y_ref[...]) * dinv[:, None] + b_ref[...]


def kernel(x, edge_index, W1, b1, W2, b2):
    src = edge_index[0].astype(jnp.int32)
    dst = edge_index[1].astype(jnp.int32)
    pad = jnp.full((EP - E,), N, jnp.int32)
    srcp = jnp.concatenate([src, pad])
    dstp = jnp.concatenate([dst, pad])
    xp = jnp.pad(x, ((0, NP - N), (0, 0)))
    z1 = jnp.zeros((NP,), jnp.float32)
    zh = jnp.zeros((NP, D_HID), jnp.float32)
    b1r = b1.reshape(1, D_HID)
    b2r = jnp.pad(b2, (0, D_HID - D_OUT)).reshape(1, D_HID)
    W2p = jnp.pad(W2, ((0, 0), (0, D_HID - D_OUT)))

    dp = _deg(dstp, z1)

    grid = (NP // BR,)
    y1 = pl.pallas_call(
        _mm1_body,
        grid=grid,
        in_specs=[
            pl.BlockSpec((2, BR), lambda i: (0, i)),
            pl.BlockSpec((BR, D_IN), lambda i: (i, 0)),
            pl.BlockSpec((D_IN, D_HID), lambda i: (0, 0)),
        ],
        out_specs=pl.BlockSpec((BR, D_HID), lambda i: (i, 0)),
        out_shape=jax.ShapeDtypeStruct((NP, D_HID), jnp.float32),
    )(dp, xp, W1)

    a1 = _agg_hid(y1, srcp, dstp, zh)

    y2 = pl.pallas_call(
        _mm2_body,
        grid=grid,
        in_specs=[
            pl.BlockSpec((2, BR), lambda i: (0, i)),
            pl.BlockSpec((2, BR, D_HID), lambda i: (0, i, 0)),
            pl.BlockSpec((BR, D_HID), lambda i: (i, 0)),
            pl.BlockSpec((1, D_HID), lambda i: (0, 0)),
            pl.BlockSpec((D_HID, D_HID), lambda i: (0, 0)),
        ],
        out_specs=pl.BlockSpec((BR, D_HID), lambda i: (i, 0)),
        out_shape=jax.ShapeDtypeStruct((NP, D_HID), jnp.float32),
    )(dp, a1, y1, b1r, W2p)

    a2 = _agg_hid(y2, srcp, dstp, zh)

    out = pl.pallas_call(
        _fin_body,
        grid=grid,
        in_specs=[
            pl.BlockSpec((2, BR), lambda i: (0, i)),
            pl.BlockSpec((2, BR, D_HID), lambda i: (0, i, 0)),
            pl.BlockSpec((BR, D_HID), lambda i: (i, 0)),
            pl.BlockSpec((1, D_HID), lambda i: (0, 0)),
        ],
        out_specs=pl.BlockSpec((BR, D_HID), lambda i: (i, 0)),
        out_shape=jax.ShapeDtypeStruct((NP, D_HID), jnp.float32),
    )(dp, a2, y2, b2r)

    return out[:N, :D_OUT]
